# Initial kernel scaffold; baseline (speedup 1.0000x reference)
import jax
import jax.numpy as jnp
from jax.experimental import pallas as pl


def kernel(x, weight, bias, running_mean, running_var, lookup_div):
    raise NotImplementedError("write your pallas kernel here")


# speedup vs baseline: 587.9436x; 587.9436x over previous
import jax
import jax.numpy as jnp
from jax.experimental import pallas as pl
from jax.experimental.pallas import tpu as pltpu

_EPS = 1e-5
_G = 8
_HW_S = 98
_LANES = 128


def _bnorm_lut_kernel(jc_ref, mean_ref, tab_ref, x_ref, o_ref):
    i = pl.program_id(0)
    for g in range(_G):
        c = (i * _G + g) % 64
        row = tab_ref[jc_ref[c]]
        lo = jnp.broadcast_to(row[0:1, :], (_HW_S, _LANES))
        hi = jnp.broadcast_to(row[1:2, :], (_HW_S, _LANES))
        a = jnp.abs(x_ref[g] - mean_ref[c])
        idx = jnp.round(jnp.minimum(a, 255.0)).astype(jnp.int32)
        low7 = idx & 127
        vlo = jnp.take_along_axis(lo, low7, axis=1)
        vhi = jnp.take_along_axis(hi, low7, axis=1)
        o_ref[g] = jnp.where(idx < 128, vlo, vhi)


def kernel(x, weight, bias, running_mean, running_var, lookup_div):
    B, C, H, W = x.shape
    n_rows = B * C
    xr = x.reshape(n_rows, _HW_S, _LANES)
    jc = jnp.abs(
        jnp.clip(jnp.round(jnp.sqrt(running_var + _EPS)), -255.0, 255.0)
    ).astype(jnp.int32)
    tabT = lookup_div.T.reshape(256, 2, _LANES)

    out = pl.pallas_call(
        _bnorm_lut_kernel,
        grid=(n_rows // _G,),
        in_specs=[
            pl.BlockSpec(memory_space=pltpu.SMEM),
            pl.BlockSpec(memory_space=pltpu.SMEM),
            pl.BlockSpec((256, 2, _LANES), lambda i: (0, 0, 0)),
            pl.BlockSpec((_G, _HW_S, _LANES), lambda i: (i, 0, 0)),
        ],
        out_specs=pl.BlockSpec((_G, _HW_S, _LANES), lambda i: (i, 0, 0)),
        out_shape=jax.ShapeDtypeStruct((n_rows, _HW_S, _LANES), jnp.float32),
        compiler_params=pltpu.CompilerParams(
            dimension_semantics=("parallel",),
        ),
    )(jc, running_mean, tabT, xr)
    return out.reshape(B, C, H, W)


# speedup vs baseline: 1206.8552x; 2.0527x over previous
import jax
import jax.numpy as jnp
from jax.experimental import pallas as pl
from jax.experimental.pallas import tpu as pltpu

_EPS = 1e-5
_G = 8
_LANES = 128


def _bnorm_lut_kernel(jc_ref, mean_ref, tab_ref, x_ref, o_ref):
    cg = pl.program_id(1)
    H, W = x_ref.shape[2], x_ref.shape[3]
    for g in range(_G):
        c = cg * _G + g
        row = tab_ref[jc_ref[c]]
        lo = jnp.broadcast_to(row[0:1, :], (H, _LANES))
        hi = jnp.broadcast_to(row[1:2, :], (H, _LANES))
        a = jnp.abs(x_ref[0, g] - mean_ref[c])
        idx = jnp.round(jnp.minimum(a, 255.0)).astype(jnp.int32)
        low7 = idx & 127
        vlo = jnp.take_along_axis(lo, low7, axis=1)
        vhi = jnp.take_along_axis(hi, low7, axis=1)
        o_ref[0, g] = jnp.where(idx < 128, vlo, vhi)


def kernel(x, weight, bias, running_mean, running_var, lookup_div):
    B, C, H, W = x.shape
    jc = jnp.abs(
        jnp.clip(jnp.round(jnp.sqrt(running_var + _EPS)), -255.0, 255.0)
    ).astype(jnp.int32)
    tabT = lookup_div.T.reshape(256, 2, _LANES)

    return pl.pallas_call(
        _bnorm_lut_kernel,
        grid=(B, C // _G),
        in_specs=[
            pl.BlockSpec(memory_space=pltpu.SMEM),
            pl.BlockSpec(memory_space=pltpu.SMEM),
            pl.BlockSpec((256, 2, _LANES), lambda b, cg: (0, 0, 0)),
            pl.BlockSpec((1, _G, H, W), lambda b, cg: (b, cg, 0, 0)),
        ],
        out_specs=pl.BlockSpec((1, _G, H, W), lambda b, cg: (b, cg, 0, 0)),
        out_shape=jax.ShapeDtypeStruct((B, C, H, W), jnp.float32),
        compiler_params=pltpu.CompilerParams(
            dimension_semantics=("parallel", "parallel"),
        ),
    )(jc, running_mean, tabT, x)


# speedup vs baseline: 1580.4497x; 1.3096x over previous
import jax
import jax.numpy as jnp
from jax.experimental import pallas as pl
from jax.experimental.pallas import tpu as pltpu

_EPS = 1e-5
_G = 16
_LANES = 128


def _bnorm_lut_kernel(jc_ref, mean_ref, tab_ref, x_ref, o_ref):
    cg = pl.program_id(1)
    H, W = x_ref.shape[2], x_ref.shape[3]
    for g in range(_G):
        c = cg * _G + g
        row = tab_ref[jc_ref[c]]
        lo = jnp.broadcast_to(row[0:1, :], (H, _LANES))
        hi = jnp.broadcast_to(row[1:2, :], (H, _LANES))
        a = jnp.abs(x_ref[0, g] - mean_ref[c])
        idx = jnp.round(jnp.minimum(a, 255.0)).astype(jnp.int32)
        low7 = idx & 127
        vlo = jnp.take_along_axis(lo, low7, axis=1)
        vhi = jnp.take_along_axis(hi, low7, axis=1)
        o_ref[0, g] = jnp.where(idx < 128, vlo, vhi)


def kernel(x, weight, bias, running_mean, running_var, lookup_div):
    B, C, H, W = x.shape
    jc = jnp.abs(
        jnp.clip(jnp.round(jnp.sqrt(running_var + _EPS)), -255.0, 255.0)
    ).astype(jnp.int32)
    tabT = lookup_div.T.reshape(256, 2, _LANES)

    return pl.pallas_call(
        _bnorm_lut_kernel,
        grid=(B, C // _G),
        in_specs=[
            pl.BlockSpec(memory_space=pltpu.SMEM),
            pl.BlockSpec(memory_space=pltpu.SMEM),
            pl.BlockSpec((256, 2, _LANES), lambda b, cg: (0, 0, 0)),
            pl.BlockSpec((1, _G, H, W), lambda b, cg: (b, cg, 0, 0)),
        ],
        out_specs=pl.BlockSpec((1, _G, H, W), lambda b, cg: (b, cg, 0, 0)),
        out_shape=jax.ShapeDtypeStruct((B, C, H, W), jnp.float32),
        compiler_params=pltpu.CompilerParams(
            dimension_semantics=("parallel", "parallel"),
        ),
    )(jc, running_mean, tabT, x)


# speedup vs baseline: 1795.4227x; 1.1360x over previous
import jax
import jax.numpy as jnp
from jax.experimental import pallas as pl
from jax.experimental.pallas import tpu as pltpu

_EPS = 1e-5
_G = 32
_LANES = 128


def _bnorm_lut_kernel(jc_ref, mean_ref, tab_ref, x_ref, o_ref):
    cg = pl.program_id(1)
    H, W = x_ref.shape[2], x_ref.shape[3]
    for g in range(_G):
        c = cg * _G + g
        row = tab_ref[jc_ref[c]]
        lo = jnp.broadcast_to(row[0:1, :], (H, _LANES))
        hi = jnp.broadcast_to(row[1:2, :], (H, _LANES))
        a = jnp.abs(x_ref[0, g] - mean_ref[c])
        idx = jnp.round(jnp.minimum(a, 255.0)).astype(jnp.int32)
        low7 = idx & 127
        vlo = jnp.take_along_axis(lo, low7, axis=1)
        vhi = jnp.take_along_axis(hi, low7, axis=1)
        o_ref[0, g] = jnp.where(idx < 128, vlo, vhi)


def kernel(x, weight, bias, running_mean, running_var, lookup_div):
    B, C, H, W = x.shape
    jc = jnp.abs(
        jnp.clip(jnp.round(jnp.sqrt(running_var + _EPS)), -255.0, 255.0)
    ).astype(jnp.int32)
    tabT = lookup_div.T.reshape(256, 2, _LANES)

    return pl.pallas_call(
        _bnorm_lut_kernel,
        grid=(B, C // _G),
        in_specs=[
            pl.BlockSpec(memory_space=pltpu.SMEM),
            pl.BlockSpec(memory_space=pltpu.SMEM),
            pl.BlockSpec((256, 2, _LANES), lambda b, cg: (0, 0, 0)),
            pl.BlockSpec((1, _G, H, W), lambda b, cg: (b, cg, 0, 0)),
        ],
        out_specs=pl.BlockSpec((1, _G, H, W), lambda b, cg: (b, cg, 0, 0)),
        out_shape=jax.ShapeDtypeStruct((B, C, H, W), jnp.float32),
        compiler_params=pltpu.CompilerParams(
            dimension_semantics=("parallel", "parallel"),
        ),
    )(jc, running_mean, tabT, x)


# speedup vs baseline: 1821.8357x; 1.0147x over previous
import jax
import jax.numpy as jnp
from jax.experimental import pallas as pl
from jax.experimental.pallas import tpu as pltpu

_EPS = 1e-5
_G = 64
_LANES = 128


def _bnorm_lut_kernel(jc_ref, mean_ref, tab_ref, x_ref, o_ref):
    cg = pl.program_id(1)
    H, W = x_ref.shape[2], x_ref.shape[3]
    for g in range(_G):
        c = cg * _G + g
        row = tab_ref[jc_ref[c]]
        lo = jnp.broadcast_to(row[0:1, :], (H, _LANES))
        hi = jnp.broadcast_to(row[1:2, :], (H, _LANES))
        a = jnp.abs(x_ref[0, g] - mean_ref[c])
        idx = jnp.round(jnp.minimum(a, 255.0)).astype(jnp.int32)
        low7 = idx & 127
        vlo = jnp.take_along_axis(lo, low7, axis=1)
        vhi = jnp.take_along_axis(hi, low7, axis=1)
        o_ref[0, g] = jnp.where(idx < 128, vlo, vhi)


def kernel(x, weight, bias, running_mean, running_var, lookup_div):
    B, C, H, W = x.shape
    jc = jnp.abs(
        jnp.clip(jnp.round(jnp.sqrt(running_var + _EPS)), -255.0, 255.0)
    ).astype(jnp.int32)
    tabT = lookup_div.T.reshape(256, 2, _LANES)

    return pl.pallas_call(
        _bnorm_lut_kernel,
        grid=(B, C // _G),
        in_specs=[
            pl.BlockSpec(memory_space=pltpu.SMEM),
            pl.BlockSpec(memory_space=pltpu.SMEM),
            pl.BlockSpec((256, 2, _LANES), lambda b, cg: (0, 0, 0)),
            pl.BlockSpec((1, _G, H, W), lambda b, cg: (b, cg, 0, 0)),
        ],
        out_specs=pl.BlockSpec((1, _G, H, W), lambda b, cg: (b, cg, 0, 0)),
        out_shape=jax.ShapeDtypeStruct((B, C, H, W), jnp.float32),
        compiler_params=pltpu.CompilerParams(
            dimension_semantics=("parallel", "parallel"),
        ),
    )(jc, running_mean, tabT, x)
